# Initial kernel scaffold; baseline (speedup 1.0000x reference)
import jax
import jax.numpy as jnp
from jax.experimental import pallas as pl


def kernel(x, edge_index, W0, b0, gamma0, beta0, W1, b1, gamma1, beta1, W2, b2, gamma2, beta2):
    raise NotImplementedError("write your pallas kernel here")


# speedup vs baseline: 9.8188x; 9.8188x over previous
import functools

import jax
import jax.numpy as jnp
from jax import lax
from jax.experimental import pallas as pl
from jax.experimental.pallas import tpu as pltpu
from jax.experimental.pallas import tpu_sc as plsc

_N = 10000
_D = 128
_E = 320000
_NC = 2
_NS = 16
_NW = _NC * _NS
_EPT = _E // _NW
_C = 80
_NCHUNK = _EPT // _C
_RPT = 624
_TAIL = _N - _RPT * _NS

_MESH = plsc.VectorSubcoreMesh(core_axis_name="c", subcore_axis_name="s")


@functools.partial(
    pl.kernel,
    out_type=jax.ShapeDtypeStruct((_NC * _N, _D), jnp.float32),
    mesh=_MESH,
    scratch_types=[
        pltpu.VMEM((_C,), jnp.int32),
        pltpu.VMEM((_C, _D), jnp.float32),
        pltpu.VMEM_SHARED((_N, _D), jnp.float32),
    ],
)
def _deg_kernel(dst_hbm, zeros_hbm, ones_hbm, deg_hbm, idx_v, ones_v, deg_sh):
    c = lax.axis_index("c")
    s = lax.axis_index("s")
    wid = c * _NS + s
    pltpu.sync_copy(zeros_hbm.at[pl.ds(0, _RPT)], deg_sh.at[pl.ds(s * _RPT, _RPT)])

    @pl.when(s == _NS - 1)
    def _():
        pltpu.sync_copy(zeros_hbm.at[pl.ds(0, _TAIL)],
                        deg_sh.at[pl.ds(_RPT * _NS, _TAIL)])

    pltpu.sync_copy(ones_hbm, ones_v)
    plsc.subcore_barrier()

    base = wid * _EPT

    def body(k, carry):
        off = pl.multiple_of(base + k * _C, 8)
        pltpu.sync_copy(dst_hbm.at[pl.ds(off, _C)], idx_v)
        pltpu.sync_copy(ones_v, deg_sh.at[idx_v], add=True)
        return carry

    lax.fori_loop(0, _NCHUNK, body, 0)
    plsc.subcore_barrier()

    out_base = c * _N
    pltpu.sync_copy(deg_sh.at[pl.ds(s * _RPT, _RPT)],
                    deg_hbm.at[pl.ds(out_base + s * _RPT, _RPT)])

    @pl.when(s == _NS - 1)
    def _():
        pltpu.sync_copy(deg_sh.at[pl.ds(_RPT * _NS, _TAIL)],
                        deg_hbm.at[pl.ds(out_base + _RPT * _NS, _TAIL)])


@functools.partial(
    pl.kernel,
    out_type=jax.ShapeDtypeStruct((_NC * _N, _D), jnp.float32),
    mesh=_MESH,
    scratch_types=[
        pltpu.VMEM((_C,), jnp.int32),
        pltpu.VMEM((_C,), jnp.int32),
        pltpu.VMEM((_C, _D), jnp.float32),
        pltpu.VMEM_SHARED((_N, _D), jnp.float32),
        pltpu.SemaphoreType.DMA,
    ],
)
def _scatter_kernel(g_hbm, src_hbm, dst_hbm, zeros_hbm, acc_hbm,
                    src_v, dst_v, rows_v, acc_sh, sem):
    c = lax.axis_index("c")
    s = lax.axis_index("s")
    wid = c * _NS + s
    pltpu.sync_copy(zeros_hbm.at[pl.ds(0, _RPT)], acc_sh.at[pl.ds(s * _RPT, _RPT)])

    @pl.when(s == _NS - 1)
    def _():
        pltpu.sync_copy(zeros_hbm.at[pl.ds(0, _TAIL)],
                        acc_sh.at[pl.ds(_RPT * _NS, _TAIL)])

    plsc.subcore_barrier()

    base = wid * _EPT

    def body(k, carry):
        off = pl.multiple_of(base + k * _C, 8)
        pltpu.sync_copy(src_hbm.at[pl.ds(off, _C)], src_v)
        pltpu.sync_copy(dst_hbm.at[pl.ds(off, _C)], dst_v)
        pltpu.async_copy(g_hbm.at[src_v], rows_v, sem).wait()
        pltpu.sync_copy(rows_v, acc_sh.at[dst_v], add=True)
        return carry

    lax.fori_loop(0, _NCHUNK, body, 0)
    plsc.subcore_barrier()

    out_base = c * _N
    pltpu.sync_copy(acc_sh.at[pl.ds(s * _RPT, _RPT)],
                    acc_hbm.at[pl.ds(out_base + s * _RPT, _RPT)])

    @pl.when(s == _NS - 1)
    def _():
        pltpu.sync_copy(acc_sh.at[pl.ds(_RPT * _NS, _TAIL)],
                        acc_hbm.at[pl.ds(out_base + _RPT * _NS, _TAIL)])


def _mm_first_body(x_ref, w_ref, deg_ref, g_ref, dinv_ref):
    deg = deg_ref[: _N, 0:1] + deg_ref[_N :, 0:1] + 1.0
    dinv = lax.rsqrt(deg)
    h = lax.dot_general(x_ref[...], w_ref[...], (((1,), (1,)), ((), ())),
                        preferred_element_type=jnp.float32,
                        precision=lax.Precision.HIGHEST)
    g_ref[...] = h * dinv
    dinv_ref[...] = dinv


_mm_first = pl.pallas_call(
    _mm_first_body,
    out_shape=[
        jax.ShapeDtypeStruct((_N, _D), jnp.float32),
        jax.ShapeDtypeStruct((_N, 1), jnp.float32),
    ],
)


def _bn(z, gamma_ref, beta_ref):
    mean = jnp.mean(z, axis=0, keepdims=True)
    zc = z - mean
    var = jnp.mean(zc * zc, axis=0, keepdims=True)
    return jnp.maximum(zc * lax.rsqrt(var + 1e-5) * gamma_ref[...] + beta_ref[...],
                       0.0)


def _post_mm_body(acc_ref, g_ref, dinv_ref, gamma_ref, beta_ref, w_ref,
                  y_ref, gnext_ref):
    z = (acc_ref[: _N] + acc_ref[_N :] + g_ref[...]) * dinv_ref[...]
    y = _bn(z, gamma_ref, beta_ref)
    y_ref[...] = y
    gnext_ref[...] = lax.dot_general(y, w_ref[...], (((1,), (1,)), ((), ())),
                                     preferred_element_type=jnp.float32,
                                     precision=lax.Precision.HIGHEST) * dinv_ref[...]


_post_mm = pl.pallas_call(
    _post_mm_body,
    out_shape=[
        jax.ShapeDtypeStruct((_N, _D), jnp.float32),
        jax.ShapeDtypeStruct((_N, _D), jnp.float32),
    ],
)


def _post_final_body(acc_ref, g_ref, dinv_ref, gamma_ref, beta_ref, y_ref):
    z = (acc_ref[: _N] + acc_ref[_N :] + g_ref[...]) * dinv_ref[...]
    y_ref[...] = _bn(z, gamma_ref, beta_ref)


_post_final = pl.pallas_call(
    _post_final_body,
    out_shape=jax.ShapeDtypeStruct((_N, _D), jnp.float32),
)


def kernel(x, edge_index, W0, b0, gamma0, beta0, W1, b1, gamma1, beta1,
           W2, b2, gamma2, beta2):
    src = jnp.asarray(edge_index[0], jnp.int32)
    dst = jnp.asarray(edge_index[1], jnp.int32)
    onesD = jnp.ones((_C, _D), jnp.float32)
    zerosD = jnp.zeros((_RPT, _D), jnp.float32)

    deg16 = _deg_kernel(dst, zerosD, onesD)
    g0, dinv = _mm_first(x, W0, deg16)

    acc0 = _scatter_kernel(g0, src, dst, zerosD)
    y1, g1 = _post_mm(acc0, g0, dinv, gamma0.reshape(1, _D),
                      beta0.reshape(1, _D), W1)

    acc1 = _scatter_kernel(g1, src, dst, zerosD)
    y2, g2 = _post_mm(acc1, g1, dinv, gamma1.reshape(1, _D),
                      beta1.reshape(1, _D), W2)

    acc2 = _scatter_kernel(g2, src, dst, zerosD)
    y3 = _post_final(acc2, g2, dinv, gamma2.reshape(1, _D),
                     beta2.reshape(1, _D))

    return jnp.concatenate([x, y1, y2, y3], axis=1)


# speedup vs baseline: 19.1953x; 1.9549x over previous
import functools

import jax
import jax.numpy as jnp
from jax import lax
from jax.experimental import pallas as pl
from jax.experimental.pallas import tpu as pltpu
from jax.experimental.pallas import tpu_sc as plsc

_N = 10000
_D = 128
_E = 320000
_NC = 2
_NS = 16
_NW = _NC * _NS
_EPT = _E // _NW
_C = 80
_NCHUNK = _EPT // _C
_RPT = 624
_TAIL = _N - _RPT * _NS

_MESH = plsc.VectorSubcoreMesh(core_axis_name="c", subcore_axis_name="s")


@functools.partial(
    pl.kernel,
    out_type=jax.ShapeDtypeStruct((_NC * _N, _D), jnp.float32),
    mesh=_MESH,
    scratch_types=[
        [pltpu.VMEM((_C,), jnp.int32)] * 4,
        pltpu.VMEM((_C, _D), jnp.float32),
        pltpu.VMEM_SHARED((_N, _D), jnp.float32),
        [pltpu.SemaphoreType.DMA] * 4,
        [pltpu.SemaphoreType.DMA] * 4,
    ],
)
def _deg_kernel(dst_hbm, zeros_hbm, ones_hbm, deg_hbm, dstb, ones_v, deg_sh,
                si, ss):
    c = lax.axis_index("c")
    s = lax.axis_index("s")
    wid = c * _NS + s
    pltpu.sync_copy(zeros_hbm.at[pl.ds(0, _RPT)], deg_sh.at[pl.ds(s * _RPT, _RPT)])

    @pl.when(s == _NS - 1)
    def _():
        pltpu.sync_copy(zeros_hbm.at[pl.ds(0, _TAIL)],
                        deg_sh.at[pl.ds(_RPT * _NS, _TAIL)])

    pltpu.sync_copy(ones_hbm, ones_v)
    base = wid * _EPT

    def idxload(k, j):
        pltpu.async_copy(dst_hbm.at[pl.ds(pl.multiple_of(base + k * _C, 8), _C)],
                         dstb[j], si[j])

    def idxwait(j):
        pltpu.make_async_copy(dst_hbm.at[pl.ds(0, _C)], dstb[j], si[j]).wait()

    def scatter(j, sem_j):
        pltpu.async_copy(ones_v, deg_sh.at[dstb[j]], sem_j, add=True)

    def scatter_wait(j, sem_j):
        pltpu.make_async_copy(ones_v, deg_sh.at[dstb[j]], sem_j).wait()

    plsc.subcore_barrier()

    for j in range(4):
        idxload(j, j)

    def body(i, carry):
        k = i * 4
        for j in range(4):
            idxwait(j)
            scatter(j, ss[j])
        for j in range(4):
            scatter_wait(j, ss[j])
            if j == 0:
                idxload(k + 4 + j, j)
            else:
                @pl.when(i < _NCHUNK // 4 - 1)
                def _():
                    idxload(k + 4 + j, j)
        return carry

    lax.fori_loop(0, _NCHUNK // 4, body, 0)
    idxwait(0)
    scatter(0, ss[0])
    scatter_wait(0, ss[0])
    plsc.subcore_barrier()

    out_base = c * _N
    pltpu.sync_copy(deg_sh.at[pl.ds(s * _RPT, _RPT)],
                    deg_hbm.at[pl.ds(out_base + s * _RPT, _RPT)])

    @pl.when(s == _NS - 1)
    def _():
        pltpu.sync_copy(deg_sh.at[pl.ds(_RPT * _NS, _TAIL)],
                        deg_hbm.at[pl.ds(out_base + _RPT * _NS, _TAIL)])


@functools.partial(
    pl.kernel,
    out_type=jax.ShapeDtypeStruct((_NC * _N, _D), jnp.float32),
    mesh=_MESH,
    scratch_types=[
        [pltpu.VMEM((_C,), jnp.int32)] * 4,
        [pltpu.VMEM((_C,), jnp.int32)] * 4,
        [pltpu.VMEM((_C, _D), jnp.float32)] * 4,
        [pltpu.SemaphoreType.DMA] * 4,
        [pltpu.SemaphoreType.DMA] * 4,
        [pltpu.SemaphoreType.DMA] * 4,
        pltpu.VMEM_SHARED((_N, _D), jnp.float32),
    ],
)
def _scatter_kernel(g_hbm, src_hbm, dst_hbm, zeros_hbm, acc_hbm,
                    srcb, dstb, rows, si, sg, ss, acc_sh):
    c = lax.axis_index("c")
    s = lax.axis_index("s")
    wid = c * _NS + s
    pltpu.sync_copy(zeros_hbm.at[pl.ds(0, _RPT)], acc_sh.at[pl.ds(s * _RPT, _RPT)])

    @pl.when(s == _NS - 1)
    def _():
        pltpu.sync_copy(zeros_hbm.at[pl.ds(0, _TAIL)],
                        acc_sh.at[pl.ds(_RPT * _NS, _TAIL)])

    base = wid * _EPT

    def idxload(k, j):
        off = pl.multiple_of(base + k * _C, 8)
        pltpu.async_copy(src_hbm.at[pl.ds(off, _C)], srcb[j], si[j])
        pltpu.async_copy(dst_hbm.at[pl.ds(off, _C)], dstb[j], si[j])

    def idxwait(j):
        pltpu.make_async_copy(src_hbm.at[pl.ds(0, _C)], srcb[j], si[j]).wait()
        pltpu.make_async_copy(dst_hbm.at[pl.ds(0, _C)], dstb[j], si[j]).wait()

    def gather(j):
        pltpu.async_copy(g_hbm.at[srcb[j]], rows[j], sg[j])

    def gather_wait(j):
        pltpu.make_async_copy(g_hbm.at[srcb[j]], rows[j], sg[j]).wait()

    def scatter(j):
        pltpu.async_copy(rows[j], acc_sh.at[dstb[j]], ss[j], add=True)

    def scatter_wait(j):
        pltpu.make_async_copy(rows[j], acc_sh.at[dstb[j]], ss[j]).wait()

    plsc.subcore_barrier()

    for j in range(4):
        idxload(j, j)

    for j in range(4):
        idxwait(j)
        gather(j)

    def body(i, carry):
        k = i * 4
        for j in range(4):
            gather_wait(j)
            scatter(j)
        for j in range(4):
            scatter_wait(j)
            if j == 0:
                idxload(k + 4 + j, j)
            else:
                @pl.when(i < _NCHUNK // 4 - 1)
                def _():
                    idxload(k + 4 + j, j)
        for j in range(4):
            if j == 0:
                idxwait(j)
                gather(j)
            else:
                @pl.when(i < _NCHUNK // 4 - 1)
                def _():
                    idxwait(j)
                    gather(j)
        return carry

    lax.fori_loop(0, _NCHUNK // 4, body, 0)
    gather_wait(0)
    scatter(0)
    scatter_wait(0)
    plsc.subcore_barrier()

    out_base = c * _N
    pltpu.sync_copy(acc_sh.at[pl.ds(s * _RPT, _RPT)],
                    acc_hbm.at[pl.ds(out_base + s * _RPT, _RPT)])

    @pl.when(s == _NS - 1)
    def _():
        pltpu.sync_copy(acc_sh.at[pl.ds(_RPT * _NS, _TAIL)],
                        acc_hbm.at[pl.ds(out_base + _RPT * _NS, _TAIL)])


def _mm_first_body(x_ref, w_ref, deg_ref, g_ref, dinv_ref):
    deg = deg_ref[: _N, 0:1] + deg_ref[_N :, 0:1] + 1.0
    dinv = lax.rsqrt(deg)
    h = lax.dot_general(x_ref[...], w_ref[...], (((1,), (1,)), ((), ())),
                        preferred_element_type=jnp.float32,
                        precision=lax.Precision.HIGHEST)
    g_ref[...] = h * dinv
    dinv_ref[...] = dinv


_mm_first = pl.pallas_call(
    _mm_first_body,
    out_shape=[
        jax.ShapeDtypeStruct((_N, _D), jnp.float32),
        jax.ShapeDtypeStruct((_N, 1), jnp.float32),
    ],
)


def _bn(z, gamma_ref, beta_ref):
    mean = jnp.mean(z, axis=0, keepdims=True)
    zc = z - mean
    var = jnp.mean(zc * zc, axis=0, keepdims=True)
    return jnp.maximum(zc * lax.rsqrt(var + 1e-5) * gamma_ref[...] + beta_ref[...],
                       0.0)


def _post_mm_body(acc_ref, g_ref, dinv_ref, gamma_ref, beta_ref, w_ref,
                  y_ref, gnext_ref):
    z = (acc_ref[: _N] + acc_ref[_N :] + g_ref[...]) * dinv_ref[...]
    y = _bn(z, gamma_ref, beta_ref)
    y_ref[...] = y
    gnext_ref[...] = lax.dot_general(y, w_ref[...], (((1,), (1,)), ((), ())),
                                     preferred_element_type=jnp.float32,
                                     precision=lax.Precision.HIGHEST) * dinv_ref[...]


_post_mm = pl.pallas_call(
    _post_mm_body,
    out_shape=[
        jax.ShapeDtypeStruct((_N, _D), jnp.float32),
        jax.ShapeDtypeStruct((_N, _D), jnp.float32),
    ],
)


def _post_final_body(acc_ref, g_ref, dinv_ref, gamma_ref, beta_ref, y_ref):
    z = (acc_ref[: _N] + acc_ref[_N :] + g_ref[...]) * dinv_ref[...]
    y_ref[...] = _bn(z, gamma_ref, beta_ref)


_post_final = pl.pallas_call(
    _post_final_body,
    out_shape=jax.ShapeDtypeStruct((_N, _D), jnp.float32),
)


def kernel(x, edge_index, W0, b0, gamma0, beta0, W1, b1, gamma1, beta1,
           W2, b2, gamma2, beta2):
    src = jnp.asarray(edge_index[0], jnp.int32)
    dst = jnp.asarray(edge_index[1], jnp.int32)
    onesD = jnp.ones((_C, _D), jnp.float32)
    zerosD = jnp.zeros((_RPT, _D), jnp.float32)

    deg16 = _deg_kernel(dst, zerosD, onesD)
    g0, dinv = _mm_first(x, W0, deg16)

    acc0 = _scatter_kernel(g0, src, dst, zerosD)
    y1, g1 = _post_mm(acc0, g0, dinv, gamma0.reshape(1, _D),
                      beta0.reshape(1, _D), W1)

    acc1 = _scatter_kernel(g1, src, dst, zerosD)
    y2, g2 = _post_mm(acc1, g1, dinv, gamma1.reshape(1, _D),
                      beta1.reshape(1, _D), W2)

    acc2 = _scatter_kernel(g2, src, dst, zerosD)
    y3 = _post_final(acc2, g2, dinv, gamma2.reshape(1, _D),
                     beta2.reshape(1, _D))

    return jnp.concatenate([x, y1, y2, y3], axis=1)


# speedup vs baseline: 23.3824x; 1.2181x over previous
import functools

import jax
import jax.numpy as jnp
from jax import lax
from jax.experimental import pallas as pl
from jax.experimental.pallas import tpu as pltpu
from jax.experimental.pallas import tpu_sc as plsc

_N = 10000
_D = 128
_E = 320000
_NC = 2
_NS = 16
_NW = _NC * _NS
_EPT = _E // _NW
_C = 80
_NCHUNK = _EPT // _C
_RPT = 624
_TAIL = _N - _RPT * _NS

_MESH = plsc.VectorSubcoreMesh(core_axis_name="c", subcore_axis_name="s")


@functools.partial(
    pl.kernel,
    out_type=jax.ShapeDtypeStruct((_NC * _N, _D), jnp.float32),
    mesh=_MESH,
    scratch_types=[
        [pltpu.VMEM((_C,), jnp.int32)] * 4,
        pltpu.VMEM((_C, _D), jnp.float32),
        pltpu.VMEM_SHARED((_N, _D), jnp.float32),
        [pltpu.SemaphoreType.DMA] * 4,
        [pltpu.SemaphoreType.DMA] * 4,
    ],
)
def _deg_kernel(dst_hbm, zeros_hbm, ones_hbm, deg_hbm, dstb, ones_v, deg_sh,
                si, ss):
    c = lax.axis_index("c")
    s = lax.axis_index("s")
    wid = c * _NS + s
    pltpu.sync_copy(zeros_hbm.at[pl.ds(0, _RPT)], deg_sh.at[pl.ds(s * _RPT, _RPT)])

    @pl.when(s == _NS - 1)
    def _():
        pltpu.sync_copy(zeros_hbm.at[pl.ds(0, _TAIL)],
                        deg_sh.at[pl.ds(_RPT * _NS, _TAIL)])

    pltpu.sync_copy(ones_hbm, ones_v)
    base = wid * _EPT

    def idxload(k, j):
        pltpu.async_copy(dst_hbm.at[pl.ds(pl.multiple_of(base + k * _C, 8), _C)],
                         dstb[j], si[j])

    def idxwait(j):
        pltpu.make_async_copy(dst_hbm.at[pl.ds(0, _C)], dstb[j], si[j]).wait()

    def scatter(j, sem_j):
        pltpu.async_copy(ones_v, deg_sh.at[dstb[j]], sem_j, add=True)

    def scatter_wait(j, sem_j):
        pltpu.make_async_copy(ones_v, deg_sh.at[dstb[j]], sem_j).wait()

    plsc.subcore_barrier()

    for j in range(4):
        idxload(j, j)

    def body(i, carry):
        k = i * 4
        for j in range(4):
            idxwait(j)
            scatter(j, ss[j])
        for j in range(4):
            scatter_wait(j, ss[j])
            if j == 0:
                idxload(k + 4 + j, j)
            else:
                @pl.when(i < _NCHUNK // 4 - 1)
                def _():
                    idxload(k + 4 + j, j)
        return carry

    lax.fori_loop(0, _NCHUNK // 4, body, 0)
    idxwait(0)
    scatter(0, ss[0])
    scatter_wait(0, ss[0])
    plsc.subcore_barrier()

    out_base = c * _N
    pltpu.sync_copy(deg_sh.at[pl.ds(s * _RPT, _RPT)],
                    deg_hbm.at[pl.ds(out_base + s * _RPT, _RPT)])

    @pl.when(s == _NS - 1)
    def _():
        pltpu.sync_copy(deg_sh.at[pl.ds(_RPT * _NS, _TAIL)],
                        deg_hbm.at[pl.ds(out_base + _RPT * _NS, _TAIL)])


@functools.partial(
    pl.kernel,
    out_type=jax.ShapeDtypeStruct((_NC * _N, _D), jnp.float32),
    mesh=_MESH,
    scratch_types=[
        [pltpu.VMEM((_C,), jnp.int32)] * 8,
        [pltpu.VMEM((_C,), jnp.int32)] * 8,
        [pltpu.VMEM((_C, _D), jnp.float32)] * 4,
        [pltpu.SemaphoreType.DMA] * 8,
        [pltpu.SemaphoreType.DMA] * 4,
        [pltpu.SemaphoreType.DMA] * 4,
        pltpu.VMEM_SHARED((_N, _D), jnp.float32),
    ],
)
def _scatter_kernel(g_hbm, src_hbm, dst_hbm, zeros_hbm, acc_hbm,
                    srcb, dstb, rows, si, sg, ss, acc_sh):
    c = lax.axis_index("c")
    s = lax.axis_index("s")
    wid = c * _NS + s
    pltpu.sync_copy(zeros_hbm.at[pl.ds(0, _RPT)], acc_sh.at[pl.ds(s * _RPT, _RPT)])

    @pl.when(s == _NS - 1)
    def _():
        pltpu.sync_copy(zeros_hbm.at[pl.ds(0, _TAIL)],
                        acc_sh.at[pl.ds(_RPT * _NS, _TAIL)])

    base = wid * _EPT

    def idxload(u, js):
        off = pl.multiple_of(base + u * _C, 8)
        pltpu.async_copy(src_hbm.at[pl.ds(off, _C)], srcb[js], si[js])
        pltpu.async_copy(dst_hbm.at[pl.ds(off, _C)], dstb[js], si[js])

    def idxwait(js):
        pltpu.make_async_copy(src_hbm.at[pl.ds(0, _C)], srcb[js], si[js]).wait()
        pltpu.make_async_copy(dst_hbm.at[pl.ds(0, _C)], dstb[js], si[js]).wait()

    def gather(js, jr):
        pltpu.async_copy(g_hbm.at[srcb[js]], rows[jr], sg[jr])

    def gather_wait(js, jr):
        pltpu.make_async_copy(g_hbm.at[srcb[js]], rows[jr], sg[jr]).wait()

    def scatter(js, jr):
        pltpu.async_copy(rows[jr], acc_sh.at[dstb[js]], ss[jr], add=True)

    def scatter_wait(js, jr):
        pltpu.make_async_copy(rows[jr], acc_sh.at[dstb[js]], ss[jr]).wait()

    plsc.subcore_barrier()

    n_rounds = (_NCHUNK + 6 + 7) // 8

    def guarded(i, j, delta, fn):
        lo = max(0, -((j - delta) // 8))
        hi = (_NCHUNK - 1 + delta - j) // 8
        if lo == 0 and hi >= n_rounds - 1:
            fn()
        else:
            @pl.when((i >= lo) & (i <= hi))
            def _():
                fn()

    def body(i, carry):
        for j in range(8):
            t = i * 8 + j
            guarded(i, j, 0, lambda t=t, j=j: idxload(t, j))
            guarded(i, j, 6, lambda j=j: scatter_wait((j + 2) % 8, (j + 2) % 4))
            guarded(i, j, 2, lambda j=j: idxwait((j + 6) % 8))
            guarded(i, j, 2, lambda j=j: gather((j + 6) % 8, (j + 2) % 4))
            guarded(i, j, 4, lambda j=j: gather_wait((j + 4) % 8, j % 4))
            guarded(i, j, 4, lambda j=j: scatter((j + 4) % 8, j % 4))
        return carry

    lax.fori_loop(0, n_rounds, body, 0)
    plsc.subcore_barrier()

    out_base = c * _N
    pltpu.sync_copy(acc_sh.at[pl.ds(s * _RPT, _RPT)],
                    acc_hbm.at[pl.ds(out_base + s * _RPT, _RPT)])

    @pl.when(s == _NS - 1)
    def _():
        pltpu.sync_copy(acc_sh.at[pl.ds(_RPT * _NS, _TAIL)],
                        acc_hbm.at[pl.ds(out_base + _RPT * _NS, _TAIL)])


def _mm_first_body(x_ref, w_ref, deg_ref, g_ref, dinv_ref):
    deg = deg_ref[: _N, 0:1] + deg_ref[_N :, 0:1] + 1.0
    dinv = lax.rsqrt(deg)
    h = lax.dot_general(x_ref[...], w_ref[...], (((1,), (1,)), ((), ())),
                        preferred_element_type=jnp.float32,
                        precision=lax.Precision.HIGHEST)
    g_ref[...] = h * dinv
    dinv_ref[...] = dinv


_mm_first = pl.pallas_call(
    _mm_first_body,
    out_shape=[
        jax.ShapeDtypeStruct((_N, _D), jnp.float32),
        jax.ShapeDtypeStruct((_N, 1), jnp.float32),
    ],
)


def _bn(z, gamma_ref, beta_ref):
    mean = jnp.mean(z, axis=0, keepdims=True)
    zc = z - mean
    var = jnp.mean(zc * zc, axis=0, keepdims=True)
    return jnp.maximum(zc * lax.rsqrt(var + 1e-5) * gamma_ref[...] + beta_ref[...],
                       0.0)


def _post_mm_body(acc_ref, g_ref, dinv_ref, gamma_ref, beta_ref, w_ref,
                  y_ref, gnext_ref):
    z = (acc_ref[: _N] + acc_ref[_N :] + g_ref[...]) * dinv_ref[...]
    y = _bn(z, gamma_ref, beta_ref)
    y_ref[...] = y
    gnext_ref[...] = lax.dot_general(y, w_ref[...], (((1,), (1,)), ((), ())),
                                     preferred_element_type=jnp.float32,
                                     precision=lax.Precision.HIGHEST) * dinv_ref[...]


_post_mm = pl.pallas_call(
    _post_mm_body,
    out_shape=[
        jax.ShapeDtypeStruct((_N, _D), jnp.float32),
        jax.ShapeDtypeStruct((_N, _D), jnp.float32),
    ],
)


def _post_final_body(acc_ref, g_ref, dinv_ref, gamma_ref, beta_ref, y_ref):
    z = (acc_ref[: _N] + acc_ref[_N :] + g_ref[...]) * dinv_ref[...]
    y_ref[...] = _bn(z, gamma_ref, beta_ref)


_post_final = pl.pallas_call(
    _post_final_body,
    out_shape=jax.ShapeDtypeStruct((_N, _D), jnp.float32),
)


def kernel(x, edge_index, W0, b0, gamma0, beta0, W1, b1, gamma1, beta1,
           W2, b2, gamma2, beta2):
    src = jnp.asarray(edge_index[0], jnp.int32)
    dst = jnp.asarray(edge_index[1], jnp.int32)
    onesD = jnp.ones((_C, _D), jnp.float32)
    zerosD = jnp.zeros((_RPT, _D), jnp.float32)

    deg16 = _deg_kernel(dst, zerosD, onesD)
    g0, dinv = _mm_first(x, W0, deg16)

    acc0 = _scatter_kernel(g0, src, dst, zerosD)
    y1, g1 = _post_mm(acc0, g0, dinv, gamma0.reshape(1, _D),
                      beta0.reshape(1, _D), W1)

    acc1 = _scatter_kernel(g1, src, dst, zerosD)
    y2, g2 = _post_mm(acc1, g1, dinv, gamma1.reshape(1, _D),
                      beta1.reshape(1, _D), W2)

    acc2 = _scatter_kernel(g2, src, dst, zerosD)
    y3 = _post_final(acc2, g2, dinv, gamma2.reshape(1, _D),
                     beta2.reshape(1, _D))

    return jnp.concatenate([x, y1, y2, y3], axis=1)


# speedup vs baseline: 23.5126x; 1.0056x over previous
import functools

import jax
import jax.numpy as jnp
from jax import lax
from jax.experimental import pallas as pl
from jax.experimental.pallas import tpu as pltpu
from jax.experimental.pallas import tpu_sc as plsc

_N = 10000
_D = 128
_E = 320000
_NC = 2
_NS = 16
_NW = _NC * _NS
_EPT = _E // _NW
_C = 80
_NCHUNK = _EPT // _C
_RPT = 624
_TAIL = _N - _RPT * _NS

_MESH = plsc.VectorSubcoreMesh(core_axis_name="c", subcore_axis_name="s")


@functools.partial(
    pl.kernel,
    out_type=jax.ShapeDtypeStruct((_NC * _N, _D), jnp.float32),
    mesh=_MESH,
    scratch_types=[
        [pltpu.VMEM((_C,), jnp.int32)] * 4,
        pltpu.VMEM((_C, _D), jnp.float32),
        pltpu.VMEM_SHARED((_N, _D), jnp.float32),
        [pltpu.SemaphoreType.DMA] * 4,
        [pltpu.SemaphoreType.DMA] * 4,
    ],
)
def _deg_kernel(dst_hbm, zeros_hbm, ones_hbm, deg_hbm, dstb, ones_v, deg_sh,
                si, ss):
    c = lax.axis_index("c")
    s = lax.axis_index("s")
    wid = c * _NS + s
    pltpu.sync_copy(zeros_hbm.at[pl.ds(0, _RPT)], deg_sh.at[pl.ds(s * _RPT, _RPT)])

    @pl.when(s == _NS - 1)
    def _():
        pltpu.sync_copy(zeros_hbm.at[pl.ds(0, _TAIL)],
                        deg_sh.at[pl.ds(_RPT * _NS, _TAIL)])

    pltpu.sync_copy(ones_hbm, ones_v)
    base = wid * _EPT

    def idxload(k, j):
        pltpu.async_copy(dst_hbm.at[pl.ds(pl.multiple_of(base + k * _C, 8), _C)],
                         dstb[j], si[j])

    def idxwait(j):
        pltpu.make_async_copy(dst_hbm.at[pl.ds(0, _C)], dstb[j], si[j]).wait()

    def scatter(j, sem_j):
        pltpu.async_copy(ones_v, deg_sh.at[dstb[j]], sem_j, add=True)

    def scatter_wait(j, sem_j):
        pltpu.make_async_copy(ones_v, deg_sh.at[dstb[j]], sem_j).wait()

    plsc.subcore_barrier()

    for j in range(4):
        idxload(j, j)

    def body(i, carry):
        k = i * 4
        for j in range(4):
            idxwait(j)
            scatter(j, ss[j])
        for j in range(4):
            scatter_wait(j, ss[j])
            if j == 0:
                idxload(k + 4 + j, j)
            else:
                @pl.when(i < _NCHUNK // 4 - 1)
                def _():
                    idxload(k + 4 + j, j)
        return carry

    lax.fori_loop(0, _NCHUNK // 4, body, 0)
    idxwait(0)
    scatter(0, ss[0])
    scatter_wait(0, ss[0])
    plsc.subcore_barrier()

    out_base = c * _N
    pltpu.sync_copy(deg_sh.at[pl.ds(s * _RPT, _RPT)],
                    deg_hbm.at[pl.ds(out_base + s * _RPT, _RPT)])

    @pl.when(s == _NS - 1)
    def _():
        pltpu.sync_copy(deg_sh.at[pl.ds(_RPT * _NS, _TAIL)],
                        deg_hbm.at[pl.ds(out_base + _RPT * _NS, _TAIL)])


@functools.partial(
    pl.kernel,
    out_type=jax.ShapeDtypeStruct((_NC * _N, _D), jnp.float32),
    mesh=_MESH,
    scratch_types=[
        [pltpu.VMEM((_C,), jnp.int32)] * 8,
        [pltpu.VMEM((_C,), jnp.int32)] * 8,
        [pltpu.VMEM((_C, _D), jnp.float32)] * 4,
        [pltpu.SemaphoreType.DMA] * 8,
        [pltpu.SemaphoreType.DMA] * 4,
        [pltpu.SemaphoreType.DMA] * 4,
        pltpu.VMEM_SHARED((_N, _D), jnp.float32),
    ],
)
def _scatter_kernel(g_hbm, src_hbm, dst_hbm, zeros_hbm, acc_hbm,
                    srcb, dstb, rows, si, sg, ss, acc_sh):
    c = lax.axis_index("c")
    s = lax.axis_index("s")
    wid = c * _NS + s
    pltpu.sync_copy(zeros_hbm.at[pl.ds(0, _RPT)], acc_sh.at[pl.ds(s * _RPT, _RPT)])

    @pl.when(s == _NS - 1)
    def _():
        pltpu.sync_copy(zeros_hbm.at[pl.ds(0, _TAIL)],
                        acc_sh.at[pl.ds(_RPT * _NS, _TAIL)])

    base = wid * _EPT

    def idxload(u, js):
        off = pl.multiple_of(base + u * _C, 8)
        pltpu.async_copy(src_hbm.at[pl.ds(off, _C)], srcb[js], si[js])
        pltpu.async_copy(dst_hbm.at[pl.ds(off, _C)], dstb[js], si[js])

    def idxwait(js):
        pltpu.make_async_copy(src_hbm.at[pl.ds(0, _C)], srcb[js], si[js]).wait()
        pltpu.make_async_copy(dst_hbm.at[pl.ds(0, _C)], dstb[js], si[js]).wait()

    def gather(js, jr):
        pltpu.async_copy(g_hbm.at[srcb[js]], rows[jr], sg[jr])

    def gather_wait(js, jr):
        pltpu.make_async_copy(g_hbm.at[srcb[js]], rows[jr], sg[jr]).wait()

    def scatter(js, jr):
        pltpu.async_copy(rows[jr], acc_sh.at[dstb[js]], ss[jr], add=True)

    def scatter_wait(js, jr):
        pltpu.make_async_copy(rows[jr], acc_sh.at[dstb[js]], ss[jr]).wait()

    plsc.subcore_barrier()

    n_rounds = (_NCHUNK + 6 + 7) // 8

    def guarded(i, j, delta, fn):
        lo = max(0, -((j - delta) // 8))
        hi = (_NCHUNK - 1 + delta - j) // 8
        if lo == 0 and hi >= n_rounds - 1:
            fn()
        else:
            @pl.when((i >= lo) & (i <= hi))
            def _():
                fn()

    def body(i, carry):
        for j in range(8):
            t = i * 8 + j
            guarded(i, j, 0, lambda t=t, j=j: idxload(t, j))
            guarded(i, j, 6, lambda j=j: scatter_wait((j + 2) % 8, (j + 2) % 4))
            guarded(i, j, 2, lambda j=j: idxwait((j + 6) % 8))
            guarded(i, j, 2, lambda j=j: gather((j + 6) % 8, (j + 2) % 4))
            guarded(i, j, 4, lambda j=j: gather_wait((j + 4) % 8, j % 4))
            guarded(i, j, 4, lambda j=j: scatter((j + 4) % 8, j % 4))
        return carry

    lax.fori_loop(0, n_rounds, body, 0)
    plsc.subcore_barrier()

    out_base = c * _N
    pltpu.sync_copy(acc_sh.at[pl.ds(s * _RPT, _RPT)],
                    acc_hbm.at[pl.ds(out_base + s * _RPT, _RPT)])

    @pl.when(s == _NS - 1)
    def _():
        pltpu.sync_copy(acc_sh.at[pl.ds(_RPT * _NS, _TAIL)],
                        acc_hbm.at[pl.ds(out_base + _RPT * _NS, _TAIL)])


def _mm_h0_body(x_ref, w_ref, h_ref):
    h_ref[...] = lax.dot_general(x_ref[...], w_ref[...], (((1,), (1,)), ((), ())),
                                 preferred_element_type=jnp.float32,
                                 precision=lax.Precision.HIGHEST)


_mm_h0 = pl.pallas_call(
    _mm_h0_body,
    out_shape=jax.ShapeDtypeStruct((_N, _D), jnp.float32),
)


def _scale_first_body(h_ref, deg_ref, g_ref, dinv_ref):
    deg = deg_ref[: _N, 0:1] + deg_ref[_N :, 0:1] + 1.0
    dinv = lax.rsqrt(deg)
    g_ref[...] = h_ref[...] * dinv
    dinv_ref[...] = dinv


_scale_first = pl.pallas_call(
    _scale_first_body,
    out_shape=[
        jax.ShapeDtypeStruct((_N, _D), jnp.float32),
        jax.ShapeDtypeStruct((_N, 1), jnp.float32),
    ],
)


def _bn(z, gamma_ref, beta_ref):
    mean = jnp.mean(z, axis=0, keepdims=True)
    zc = z - mean
    var = jnp.mean(zc * zc, axis=0, keepdims=True)
    return jnp.maximum(zc * lax.rsqrt(var + 1e-5) * gamma_ref[...] + beta_ref[...],
                       0.0)


def _post_mm_body(acc_ref, g_ref, dinv_ref, gamma_ref, beta_ref, w_ref,
                  y_ref, gnext_ref):
    z = (acc_ref[: _N] + acc_ref[_N :] + g_ref[...]) * dinv_ref[...]
    y = _bn(z, gamma_ref, beta_ref)
    y_ref[...] = y
    gnext_ref[...] = lax.dot_general(y, w_ref[...], (((1,), (1,)), ((), ())),
                                     preferred_element_type=jnp.float32,
                                     precision=lax.Precision.HIGHEST) * dinv_ref[...]


_post_mm = pl.pallas_call(
    _post_mm_body,
    out_shape=[
        jax.ShapeDtypeStruct((_N, _D), jnp.float32),
        jax.ShapeDtypeStruct((_N, _D), jnp.float32),
    ],
)


def _post_final_body(acc_ref, g_ref, dinv_ref, gamma_ref, beta_ref,
                     x_ref, y1_ref, y2_ref, out_ref):
    z = (acc_ref[: _N] + acc_ref[_N :] + g_ref[...]) * dinv_ref[...]
    out_ref[:, 0:_D] = x_ref[...]
    out_ref[:, _D : 2 * _D] = y1_ref[...]
    out_ref[:, 2 * _D : 3 * _D] = y2_ref[...]
    out_ref[:, 3 * _D : 4 * _D] = _bn(z, gamma_ref, beta_ref)


_post_final = pl.pallas_call(
    _post_final_body,
    out_shape=jax.ShapeDtypeStruct((_N, 4 * _D), jnp.float32),
)


def kernel(x, edge_index, W0, b0, gamma0, beta0, W1, b1, gamma1, beta1,
           W2, b2, gamma2, beta2):
    src = jnp.asarray(edge_index[0], jnp.int32)
    dst = jnp.asarray(edge_index[1], jnp.int32)
    onesD = jnp.ones((_C, _D), jnp.float32)
    zerosD = jnp.zeros((_RPT, _D), jnp.float32)

    h0 = _mm_h0(x, W0)
    deg16 = _deg_kernel(dst, zerosD, onesD)
    g0, dinv = _scale_first(h0, deg16)

    acc0 = _scatter_kernel(g0, src, dst, zerosD)
    y1, g1 = _post_mm(acc0, g0, dinv, gamma0.reshape(1, _D),
                      beta0.reshape(1, _D), W1)

    acc1 = _scatter_kernel(g1, src, dst, zerosD)
    y2, g2 = _post_mm(acc1, g1, dinv, gamma1.reshape(1, _D),
                      beta1.reshape(1, _D), W2)

    acc2 = _scatter_kernel(g2, src, dst, zerosD)
    return _post_final(acc2, g2, dinv, gamma2.reshape(1, _D),
                       beta2.reshape(1, _D), x, y1, y2)
